# baseline (speedup 1.0000x reference)
import functools

import jax
import jax.numpy as jnp
from jax import lax
from jax.experimental import pallas as pl
from jax.experimental.pallas import tpu as pltpu
from jax.experimental.pallas import tpu_sc as plsc

N, P, C, K = 8, 2048, 64, 20
NP = N * P
R = 256
NB = P // R
BIG = 3.0e38

NC, NS = 2, 16
NW = NC * NS
PW = NP // NW
CP = 32
NCHUNK = PW // CP
GW = 80
NG = CP * K // GW


def _k0_body(x_ref, wcat_ref, bb_ref, z_ref, a_ref, v_ref):
    x = x_ref[...]
    prod = jnp.dot(x, wcat_ref[...], preferred_element_type=jnp.float32)
    z_ref[...] = prod[:, 0:C]
    a_ref[...] = prod[:, C:2 * C] + bb_ref[0:1, :]
    v_ref[...] = prod[:, 2 * C:3 * C] + bb_ref[1:2, :]


def _run_k0(X, wcat, bb):
    return pl.pallas_call(
        _k0_body,
        grid=(N,),
        in_specs=[
            pl.BlockSpec((P, C), lambda n: (n, 0)),
            pl.BlockSpec((C, 3 * C), lambda n: (0, 0)),
            pl.BlockSpec((2, C), lambda n: (0, 0)),
        ],
        out_specs=[
            pl.BlockSpec((P, C), lambda n: (n, 0)),
            pl.BlockSpec((P, C), lambda n: (n, 0)),
            pl.BlockSpec((P, C), lambda n: (n, 0)),
        ],
        out_shape=[
            jax.ShapeDtypeStruct((NP, C), jnp.float32),
            jax.ShapeDtypeStruct((NP, C), jnp.float32),
            jax.ShapeDtypeStruct((NP, C), jnp.float32),
        ],
    )(X, wcat, bb)


def _k0b_body(mv_ref, w2a_ref, u_ref):
    u_ref[...] = jnp.dot(mv_ref[...], w2a_ref[...],
                         preferred_element_type=jnp.float32)


def _run_k0b(mv_ft, w2aT):
    return pl.pallas_call(
        _k0b_body,
        out_shape=jax.ShapeDtypeStruct((N, C), jnp.float32),
    )(mv_ft, w2aT)


def _k1_body(xf_ref, xr_ref, idx_ref):
    n = pl.program_id(0)
    j = pl.program_id(1)
    X = xf_ref[...]
    Xr = xr_ref[...]
    g = lax.dot_general(Xr, X, (((1,), (1,)), ((), ())),
                        preferred_element_type=jnp.float32)
    sq = jnp.sum(X * X, axis=1)
    sqr = jnp.sum(Xr * Xr, axis=1)
    adj = sqr[:, None] + sq[None, :] - 2.0 * g
    iota_c = lax.broadcasted_iota(jnp.int32, (R, P), 1)
    iota_k = lax.broadcasted_iota(jnp.int32, (R, K), 1)
    rowl = lax.broadcasted_iota(jnp.int32, (R, 1), 0) + j * R

    dval = jnp.min(jnp.where(iota_c == rowl, adj, BIG), axis=1, keepdims=True)
    idxs0 = jnp.where(iota_k == 0, rowl, 0)

    def body(k, carry):
        m_prev, idxs = carry
        masked = jnp.where(adj > m_prev, adj, BIG)
        m = jnp.min(masked, axis=1, keepdims=True)
        idx = jnp.argmin(masked, axis=1).astype(jnp.int32)[:, None]
        idxs = jnp.where(iota_k == k, idx, idxs)
        return m, idxs

    _, idxs = lax.fori_loop(1, K, body, (dval, idxs0))
    idx_ref[...] = idxs + n * P


def _run_k1(X):
    return pl.pallas_call(
        _k1_body,
        grid=(N, NB),
        in_specs=[
            pl.BlockSpec((P, C), lambda n, j: (n, 0)),
            pl.BlockSpec((R, C), lambda n, j: (n * NB + j, 0)),
        ],
        out_specs=pl.BlockSpec((R, K), lambda n, j: (n * NB + j, 0)),
        out_shape=jax.ShapeDtypeStruct((NP, K), jnp.int32),
    )(X, X)


def _k2_body(z_hbm, idx_hbm, s_hbm, idx_v, rows_v, out_v, sem):
    wid = lax.axis_index("s") * NC + lax.axis_index("c")
    p0w = wid * PW
    rows_per_gather = GW // K

    def chunk_body(ci, _):
        p0 = pl.multiple_of(p0w + ci * CP, CP)
        r0 = pl.multiple_of(p0 // rows_per_gather, 8)
        pltpu.sync_copy(idx_hbm.at[pl.ds(r0, NG)], idx_v)
        cps = [
            pltpu.async_copy(z_hbm.at[idx_v.at[j]],
                             rows_v.at[pl.ds(j * GW, GW)], sem)
            for j in range(NG)
        ]
        for cp in cps:
            cp.wait()

        def point_body(p, _):
            base = p * K
            for c4 in range(C // 16):
                sl = pl.ds(c4 * 16, 16)
                v = rows_v[base, sl]
                s1 = v
                s2 = v * v
                mx = v
                mn = v
                for k in range(1, K):
                    v = rows_v[base + k, sl]
                    s1 = s1 + v
                    s2 = s2 + v * v
                    mx = jnp.maximum(mx, v)
                    mn = jnp.minimum(mn, v)
                out_v[p, pl.ds(c4 * 16, 16)] = s1
                out_v[p, pl.ds(C + c4 * 16, 16)] = s2
                out_v[p, pl.ds(2 * C + c4 * 16, 16)] = mx
                out_v[p, pl.ds(3 * C + c4 * 16, 16)] = mn
            return 0

        lax.fori_loop(0, CP, point_body, 0)
        pltpu.sync_copy(out_v, s_hbm.at[pl.ds(p0, CP)])
        return 0

    lax.fori_loop(0, NCHUNK, chunk_body, 0)


def _run_k2(Z, idx2d):
    mesh = plsc.VectorSubcoreMesh(core_axis_name="c", subcore_axis_name="s")
    fn = pl.kernel(
        _k2_body,
        out_type=jax.ShapeDtypeStruct((NP, 4 * C), jnp.float32),
        mesh=mesh,
        scratch_types=[
            pltpu.VMEM((NG, GW), jnp.int32),
            pltpu.VMEM((CP * K, C), jnp.float32),
            pltpu.VMEM((CP, 4 * C), jnp.float32),
            pltpu.SemaphoreType.DMA,
        ],
        compiler_params=pltpu.CompilerParams(use_tc_tiling_on_sc=False),
    )
    return fn(Z, idx2d)


def _k3_body(a_ref, s_ref, v_ref, u_ref, st_ref):
    n = pl.program_id(0)
    a = a_ref[...]
    s = s_ref[...]
    v = v_ref[...]
    s1 = s[:, 0:C]
    s2 = s[:, C:2 * C]
    un = u_ref[pl.ds(n, 1), :]
    y2 = v + un
    part = jnp.concatenate([
        jnp.sum(a, axis=0, keepdims=True),
        jnp.sum(a * a, axis=0, keepdims=True),
        jnp.sum(s1, axis=0, keepdims=True),
        jnp.sum(s2, axis=0, keepdims=True),
        jnp.sum(a * s1, axis=0, keepdims=True),
        jnp.sum(y2, axis=0, keepdims=True),
        jnp.sum(y2 * y2, axis=0, keepdims=True),
        jnp.zeros((1, C), jnp.float32),
    ], axis=0)

    @pl.when(n == 0)
    def _():
        st_ref[...] = part

    @pl.when(n != 0)
    def _():
        st_ref[...] = st_ref[...] + part


def _run_k3(A, S, V, u):
    return pl.pallas_call(
        _k3_body,
        grid=(N,),
        in_specs=[
            pl.BlockSpec((P, C), lambda n: (n, 0)),
            pl.BlockSpec((P, 4 * C), lambda n: (n, 0)),
            pl.BlockSpec((P, C), lambda n: (n, 0)),
            pl.BlockSpec((N, C), lambda n: (0, 0)),
        ],
        out_specs=pl.BlockSpec((8, C), lambda n: (0, 0)),
        out_shape=jax.ShapeDtypeStruct((8, C), jnp.float32),
    )(A, S, V, u)


def _k4_body(a_ref, s_ref, v_ref, u_ref, st_ref, gb_ref, o_ref):
    n = pl.program_id(0)
    st = st_ref[...]
    fK = jnp.float32(K)
    fNPK = jnp.float32(NP * K)
    fNP = jnp.float32(NP)
    mean1 = (fK * st[0:1, :] + st[2:3, :]) / fNPK
    ey2 = (fK * st[1:2, :] + 2.0 * st[4:5, :] + st[3:4, :]) / fNPK
    var1 = ey2 - mean1 * mean1
    sc1 = gb_ref[0:1, :] * lax.rsqrt(var1 + 1e-5)
    bi1 = gb_ref[1:2, :] - mean1 * sc1
    mean2 = st[5:6, :] / fNP
    var2 = st[6:7, :] / fNP - mean2 * mean2
    sc2 = gb_ref[2:3, :] * lax.rsqrt(var2 + 1e-5)
    bi2 = gb_ref[3:4, :] - mean2 * sc2

    a = a_ref[...]
    s = s_ref[...]
    mx = s[:, 2 * C:3 * C]
    mn = s[:, 3 * C:4 * C]
    h1 = jnp.maximum((a + mx) * sc1 + bi1, 0.0)
    h2 = jnp.maximum((a + mn) * sc1 + bi1, 0.0)
    pco = jnp.maximum(h1, h2)
    un = u_ref[pl.ds(n, 1), :]
    msk = jnp.maximum((v_ref[...] + un) * sc2 + bi2, 0.0)
    o_ref[...] = pco * (1.0 + msk / (1.0 + msk))


def _run_k4(A, S, V, u, st, gb):
    return pl.pallas_call(
        _k4_body,
        grid=(N,),
        in_specs=[
            pl.BlockSpec((P, C), lambda n: (n, 0)),
            pl.BlockSpec((P, 4 * C), lambda n: (n, 0)),
            pl.BlockSpec((P, C), lambda n: (n, 0)),
            pl.BlockSpec((N, C), lambda n: (0, 0)),
            pl.BlockSpec((8, C), lambda n: (0, 0)),
            pl.BlockSpec((4, C), lambda n: (0, 0)),
        ],
        out_specs=pl.BlockSpec((P, C), lambda n: (n, 0)),
        out_shape=jax.ShapeDtypeStruct((NP, C), jnp.float32),
    )(A, S, V, u, st, gb)


@jax.jit
def kernel(pc, mv_ft, W1, b1, g1, be1, W2, b2, g2, be2):
    pcs = jnp.squeeze(pc, axis=3)
    X = jnp.transpose(pcs, (0, 2, 1)).reshape(NP, C)
    W1a = W1[:, :C]
    W1b = W1[:, C:]
    wcat = jnp.concatenate([W1b.T, (W1a - W1b).T, W2[:, 1024:].T], axis=1)
    w2aT = W2[:, :1024].T
    bb = jnp.stack([b1, b2], axis=0)
    gb = jnp.stack([g1, be1, g2, be2], axis=0)

    Z, A, V = _run_k0(X, wcat, bb)
    u = _run_k0b(mv_ft, w2aT)
    idx = _run_k1(X)
    idx2d = idx.reshape(NP * K // GW, GW)
    S = _run_k2(Z, idx2d)
    st = _run_k3(A, S, V, u)
    out = _run_k4(A, S, V, u, st, gb)
    return jnp.transpose(out.reshape(N, P, C), (0, 2, 1))[..., None]


# speedup vs baseline: 1.3456x; 1.3456x over previous
import functools

import jax
import jax.numpy as jnp
from jax import lax
from jax.experimental import pallas as pl
from jax.experimental.pallas import tpu as pltpu
from jax.experimental.pallas import tpu_sc as plsc

N, P, C, K = 8, 2048, 64, 20
NP = N * P
R = 256
NB = P // R
BIG = 3.0e38

NC, NS = 2, 16
NW = NC * NS
PW = NP // NW
CP = 32
NCHUNK = PW // CP
GW = 80
NG = CP * K // GW


def _k0_body(x_ref, wcat_ref, bb_ref, z_ref, a_ref, v_ref):
    x = x_ref[...]
    prod = jnp.dot(x, wcat_ref[...], preferred_element_type=jnp.float32)
    z_ref[...] = prod[:, 0:C]
    a_ref[...] = prod[:, C:2 * C] + bb_ref[0:1, :]
    v_ref[...] = prod[:, 2 * C:3 * C] + bb_ref[1:2, :]


def _run_k0(X, wcat, bb):
    return pl.pallas_call(
        _k0_body,
        grid=(N,),
        in_specs=[
            pl.BlockSpec((P, C), lambda n: (n, 0)),
            pl.BlockSpec((C, 3 * C), lambda n: (0, 0)),
            pl.BlockSpec((2, C), lambda n: (0, 0)),
        ],
        out_specs=[
            pl.BlockSpec((P, C), lambda n: (n, 0)),
            pl.BlockSpec((P, C), lambda n: (n, 0)),
            pl.BlockSpec((P, C), lambda n: (n, 0)),
        ],
        out_shape=[
            jax.ShapeDtypeStruct((NP, C), jnp.float32),
            jax.ShapeDtypeStruct((NP, C), jnp.float32),
            jax.ShapeDtypeStruct((NP, C), jnp.float32),
        ],
    )(X, wcat, bb)


def _k0b_body(mv_ref, w2a_ref, u_ref):
    u_ref[...] = jnp.dot(mv_ref[...], w2a_ref[...],
                         preferred_element_type=jnp.float32)


def _run_k0b(mv_ft, w2aT):
    return pl.pallas_call(
        _k0b_body,
        out_shape=jax.ShapeDtypeStruct((N, C), jnp.float32),
    )(mv_ft, w2aT)


def _k1_body(xf_ref, xr_ref, idx_ref):
    n = pl.program_id(0)
    j = pl.program_id(1)
    X = xf_ref[...]
    Xr = xr_ref[...]
    g = lax.dot_general(Xr, X, (((1,), (1,)), ((), ())),
                        preferred_element_type=jnp.float32)
    sq = jnp.sum(X * X, axis=1)
    sqr = jnp.sum(Xr * Xr, axis=1)
    adj = sqr[:, None] + sq[None, :] - 2.0 * g
    iota_c = lax.broadcasted_iota(jnp.int32, (R, P), 1)
    iota_k = lax.broadcasted_iota(jnp.int32, (R, K), 1)
    rowl = lax.broadcasted_iota(jnp.int32, (R, 1), 0) + j * R

    dval = jnp.min(jnp.where(iota_c == rowl, adj, BIG), axis=1, keepdims=True)
    idxs0 = jnp.where(iota_k == 0, rowl, 0)

    def body(k, carry):
        m_prev, idxs = carry
        masked = jnp.where(adj > m_prev, adj, BIG)
        m = jnp.min(masked, axis=1, keepdims=True)
        cand = jnp.where(masked == m, iota_c, P)
        idx = jnp.min(cand, axis=1, keepdims=True)
        idxs = jnp.where(iota_k == k, idx, idxs)
        return m, idxs

    _, idxs = lax.fori_loop(1, K, body, (dval, idxs0))
    idx_ref[...] = idxs + n * P


def _run_k1(X):
    return pl.pallas_call(
        _k1_body,
        grid=(N, NB),
        in_specs=[
            pl.BlockSpec((P, C), lambda n, j: (n, 0)),
            pl.BlockSpec((R, C), lambda n, j: (n * NB + j, 0)),
        ],
        out_specs=pl.BlockSpec((R, K), lambda n, j: (n * NB + j, 0)),
        out_shape=jax.ShapeDtypeStruct((NP, K), jnp.int32),
    )(X, X)


def _k2_body(z_hbm, idx_hbm, s_hbm, idx_v, rows_v, out_v, sem):
    wid = lax.axis_index("s") * NC + lax.axis_index("c")
    p0w = wid * PW
    rows_per_gather = GW // K

    def chunk_body(ci, _):
        p0 = pl.multiple_of(p0w + ci * CP, CP)
        r0 = pl.multiple_of(p0 // rows_per_gather, 8)
        pltpu.sync_copy(idx_hbm.at[pl.ds(r0, NG)], idx_v)
        cps = [
            pltpu.async_copy(z_hbm.at[idx_v.at[j]],
                             rows_v.at[pl.ds(j * GW, GW)], sem)
            for j in range(NG)
        ]
        for cp in cps:
            cp.wait()

        def point_body(p, _):
            base = p * K
            for c4 in range(C // 16):
                sl = pl.ds(c4 * 16, 16)
                v = rows_v[base, sl]
                s1 = v
                s2 = v * v
                mx = v
                mn = v
                for k in range(1, K):
                    v = rows_v[base + k, sl]
                    s1 = s1 + v
                    s2 = s2 + v * v
                    mx = jnp.maximum(mx, v)
                    mn = jnp.minimum(mn, v)
                out_v[p, pl.ds(c4 * 16, 16)] = s1
                out_v[p, pl.ds(C + c4 * 16, 16)] = s2
                out_v[p, pl.ds(2 * C + c4 * 16, 16)] = mx
                out_v[p, pl.ds(3 * C + c4 * 16, 16)] = mn
            return 0

        lax.fori_loop(0, CP, point_body, 0)
        pltpu.sync_copy(out_v, s_hbm.at[pl.ds(p0, CP)])
        return 0

    lax.fori_loop(0, NCHUNK, chunk_body, 0)


def _run_k2(Z, idx2d):
    mesh = plsc.VectorSubcoreMesh(core_axis_name="c", subcore_axis_name="s")
    fn = pl.kernel(
        _k2_body,
        out_type=jax.ShapeDtypeStruct((NP, 4 * C), jnp.float32),
        mesh=mesh,
        scratch_types=[
            pltpu.VMEM((NG, GW), jnp.int32),
            pltpu.VMEM((CP * K, C), jnp.float32),
            pltpu.VMEM((CP, 4 * C), jnp.float32),
            pltpu.SemaphoreType.DMA,
        ],
        compiler_params=pltpu.CompilerParams(use_tc_tiling_on_sc=False),
    )
    return fn(Z, idx2d)


def _k3_body(a_ref, s_ref, v_ref, u_ref, st_ref):
    n = pl.program_id(0)
    a = a_ref[...]
    s = s_ref[...]
    v = v_ref[...]
    s1 = s[:, 0:C]
    s2 = s[:, C:2 * C]
    un = u_ref[pl.ds(n, 1), :]
    y2 = v + un
    part = jnp.concatenate([
        jnp.sum(a, axis=0, keepdims=True),
        jnp.sum(a * a, axis=0, keepdims=True),
        jnp.sum(s1, axis=0, keepdims=True),
        jnp.sum(s2, axis=0, keepdims=True),
        jnp.sum(a * s1, axis=0, keepdims=True),
        jnp.sum(y2, axis=0, keepdims=True),
        jnp.sum(y2 * y2, axis=0, keepdims=True),
        jnp.zeros((1, C), jnp.float32),
    ], axis=0)

    @pl.when(n == 0)
    def _():
        st_ref[...] = part

    @pl.when(n != 0)
    def _():
        st_ref[...] = st_ref[...] + part


def _run_k3(A, S, V, u):
    return pl.pallas_call(
        _k3_body,
        grid=(N,),
        in_specs=[
            pl.BlockSpec((P, C), lambda n: (n, 0)),
            pl.BlockSpec((P, 4 * C), lambda n: (n, 0)),
            pl.BlockSpec((P, C), lambda n: (n, 0)),
            pl.BlockSpec((N, C), lambda n: (0, 0)),
        ],
        out_specs=pl.BlockSpec((8, C), lambda n: (0, 0)),
        out_shape=jax.ShapeDtypeStruct((8, C), jnp.float32),
    )(A, S, V, u)


def _k4_body(a_ref, s_ref, v_ref, u_ref, st_ref, gb_ref, o_ref):
    n = pl.program_id(0)
    st = st_ref[...]
    fK = jnp.float32(K)
    fNPK = jnp.float32(NP * K)
    fNP = jnp.float32(NP)
    mean1 = (fK * st[0:1, :] + st[2:3, :]) / fNPK
    ey2 = (fK * st[1:2, :] + 2.0 * st[4:5, :] + st[3:4, :]) / fNPK
    var1 = ey2 - mean1 * mean1
    sc1 = gb_ref[0:1, :] * lax.rsqrt(var1 + 1e-5)
    bi1 = gb_ref[1:2, :] - mean1 * sc1
    mean2 = st[5:6, :] / fNP
    var2 = st[6:7, :] / fNP - mean2 * mean2
    sc2 = gb_ref[2:3, :] * lax.rsqrt(var2 + 1e-5)
    bi2 = gb_ref[3:4, :] - mean2 * sc2

    a = a_ref[...]
    s = s_ref[...]
    mx = s[:, 2 * C:3 * C]
    mn = s[:, 3 * C:4 * C]
    h1 = jnp.maximum((a + mx) * sc1 + bi1, 0.0)
    h2 = jnp.maximum((a + mn) * sc1 + bi1, 0.0)
    pco = jnp.maximum(h1, h2)
    un = u_ref[pl.ds(n, 1), :]
    msk = jnp.maximum((v_ref[...] + un) * sc2 + bi2, 0.0)
    o_ref[...] = pco * (1.0 + msk / (1.0 + msk))


def _run_k4(A, S, V, u, st, gb):
    return pl.pallas_call(
        _k4_body,
        grid=(N,),
        in_specs=[
            pl.BlockSpec((P, C), lambda n: (n, 0)),
            pl.BlockSpec((P, 4 * C), lambda n: (n, 0)),
            pl.BlockSpec((P, C), lambda n: (n, 0)),
            pl.BlockSpec((N, C), lambda n: (0, 0)),
            pl.BlockSpec((8, C), lambda n: (0, 0)),
            pl.BlockSpec((4, C), lambda n: (0, 0)),
        ],
        out_specs=pl.BlockSpec((P, C), lambda n: (n, 0)),
        out_shape=jax.ShapeDtypeStruct((NP, C), jnp.float32),
    )(A, S, V, u, st, gb)


@jax.jit
def kernel(pc, mv_ft, W1, b1, g1, be1, W2, b2, g2, be2):
    pcs = jnp.squeeze(pc, axis=3)
    X = jnp.transpose(pcs, (0, 2, 1)).reshape(NP, C)
    W1a = W1[:, :C]
    W1b = W1[:, C:]
    wcat = jnp.concatenate([W1b.T, (W1a - W1b).T, W2[:, 1024:].T], axis=1)
    w2aT = W2[:, :1024].T
    bb = jnp.stack([b1, b2], axis=0)
    gb = jnp.stack([g1, be1, g2, be2], axis=0)

    Z, A, V = _run_k0(X, wcat, bb)
    u = _run_k0b(mv_ft, w2aT)
    idx = _run_k1(X)
    idx2d = idx.reshape(NP * K // GW, GW)
    S = _run_k2(Z, idx2d)
    st = _run_k3(A, S, V, u)
    out = _run_k4(A, S, V, u, st, gb)
    return jnp.transpose(out.reshape(N, P, C), (0, 2, 1))[..., None]
